# Initial kernel scaffold; baseline (speedup 1.0000x reference)
import jax
import jax.numpy as jnp
from jax.experimental import pallas as pl


def kernel(obs, action, reward, done, returns, value, action_probs, weight, indices, starts, steps):
    raise NotImplementedError("write your pallas kernel here")


# speedup vs baseline: 44.9204x; 44.9204x over previous
import functools

import jax
import jax.numpy as jnp
from jax import lax
from jax.experimental import pallas as pl
from jax.experimental.pallas import tpu as pltpu
from jax.experimental.pallas import tpu_sc as plsc

N_TRAJ = 1024
T = 256
D_OBS = 128
N_ACT = 64
BATCH = 512
W = 64

NC, NS, L = 2, 16, 16
NW = NC * NS
BPW = BATCH // NW
ROWS_PW = BPW * W
CHUNK = 128
NCHUNK = ROWS_PW // CHUNK

_SCALAR_DTYPES = (jnp.int32, jnp.float32, jnp.int32, jnp.float32, jnp.float32,
                  jnp.float32)


def _sc_body(obs_hbm, ap_hbm, a_hbm, r_hbm, d_hbm, g_hbm, v_hbm, w_hbm,
             idx_hbm, st_hbm,
             obs_out, ap_out, a_out, r_out, d_out, g_out, v_out, w_out,
             iv, sv, idxb, ob0, ob1, ab0, ab1, rows, wins,
             so0, so1, sa0, sa1, srow):
    wid = lax.axis_index("s") * NC + lax.axis_index("c")
    b0 = wid * BPW

    pltpu.sync_copy(idx_hbm.at[pl.ds(b0, BPW)], iv)
    pltpu.sync_copy(st_hbm.at[pl.ds(b0, BPW)], sv)

    lane = lax.iota(jnp.int32, L)
    ind_v = iv[...]
    st_v = sv[...]
    base = ind_v * T + st_v
    pbase = lane * W

    def build(j, c):
        p = pbase + j
        plsc.store_scatter(idxb, [p >> 7, p & (CHUNK - 1)], base + j)
        return c
    lax.fori_loop(0, W, build, 0)

    row_cps = []
    for f_hbm, rbuf in zip((a_hbm, r_hbm, d_hbm, g_hbm, v_hbm, w_hbm), rows):
        cp = pltpu.async_copy(f_hbm.at[iv], rbuf, srow)
        row_cps.append(cp)

    obufs, osems = (ob0, ob1), (so0, so1)
    abufs, asems = (ab0, ab1), (sa0, sa1)

    def fire(k):
        i = k % 2
        co = pltpu.async_copy(obs_hbm.at[idxb.at[k]], obufs[i], osems[i])
        ca = pltpu.async_copy(ap_hbm.at[idxb.at[k]], abufs[i], asems[i])
        return co, ca

    cps = fire(0)
    for k in range(NCHUNK):
        nxt = fire(k + 1) if k + 1 < NCHUNK else None
        i = k % 2
        cps[0].wait()
        pltpu.sync_copy(obufs[i], obs_out.at[pl.ds(wid * ROWS_PW + k * CHUNK, CHUNK)])
        cps[1].wait()
        pltpu.sync_copy(abufs[i], ap_out.at[pl.ds(wid * ROWS_PW + k * CHUNK, CHUNK)])
        cps = nxt

    for cp in row_cps:
        cp.wait()

    zeros = jnp.zeros((L,), jnp.int32)

    def extract(j, c):
        col = zeros + j
        for rbuf, wbuf in zip(rows, wins):
            vals = plsc.load_gather(rbuf, [lane, st_v + j])
            plsc.store_scatter(wbuf, [lane, col], vals)
        return c
    lax.fori_loop(0, W, extract, 0)

    for wbuf, obuf in zip(wins, (a_out, r_out, d_out, g_out, v_out, w_out)):
        pltpu.sync_copy(wbuf, obuf.at[pl.ds(b0, BPW)])


@jax.jit
def _sc_sample(obs2d, ap2d, action, reward, done_i, returns, value, weight,
               indices, starts):
    mesh = plsc.VectorSubcoreMesh(core_axis_name="c", subcore_axis_name="s")
    out_type = [
        jax.ShapeDtypeStruct((BATCH * W, D_OBS), jnp.float32),
        jax.ShapeDtypeStruct((BATCH * W, N_ACT), jnp.float32),
    ] + [jax.ShapeDtypeStruct((BATCH, W), dt) for dt in _SCALAR_DTYPES]
    scratch = [
        pltpu.VMEM((BPW,), jnp.int32),
        pltpu.VMEM((BPW,), jnp.int32),
        pltpu.VMEM((NCHUNK, CHUNK), jnp.int32),
        pltpu.VMEM((CHUNK, D_OBS), jnp.float32),
        pltpu.VMEM((CHUNK, D_OBS), jnp.float32),
        pltpu.VMEM((CHUNK, N_ACT), jnp.float32),
        pltpu.VMEM((CHUNK, N_ACT), jnp.float32),
        [pltpu.VMEM((BPW, T), dt) for dt in _SCALAR_DTYPES],
        [pltpu.VMEM((BPW, W), dt) for dt in _SCALAR_DTYPES],
        pltpu.SemaphoreType.DMA,
        pltpu.SemaphoreType.DMA,
        pltpu.SemaphoreType.DMA,
        pltpu.SemaphoreType.DMA,
        pltpu.SemaphoreType.DMA,
    ]
    kfn = pl.kernel(_sc_body, out_type=out_type, mesh=mesh,
                    scratch_types=scratch,
                    compiler_params=pltpu.CompilerParams(
                        needs_layout_passes=False,
                        use_tc_tiling_on_sc=False))
    return kfn(obs2d, ap2d, action, reward, done_i, returns, value, weight,
               indices, starts)


def kernel(obs, action, reward, done, returns, value, action_probs, weight,
           indices, starts, steps):
    starts = (starts + (steps - W)).astype(jnp.int32)
    indices = indices.astype(jnp.int32)
    obs2d = obs.reshape(N_TRAJ * T, D_OBS)
    ap2d = action_probs.reshape(N_TRAJ * T, N_ACT)
    done_i = done.astype(jnp.int32)
    (obs_o, ap_o, a_o, r_o, d_o, g_o, v_o, w_o) = _sc_sample(
        obs2d, ap2d, action, reward, done_i, returns, value, weight,
        indices, starts)
    return (obs_o.reshape(BATCH, W, D_OBS), a_o, r_o, d_o.astype(jnp.bool_),
            g_o, v_o, ap_o.reshape(BATCH, W, N_ACT), w_o)
